# baseline (speedup 1.0000x reference)
import functools

import jax
import jax.numpy as jnp
from jax import lax
from jax.experimental import pallas as pl
from jax.experimental.pallas import tpu as pltpu
from jax.experimental.pallas import tpu_sc as plsc

B = 16384
D = 128
H = 8192

NC = 2
NS = 16
NW = NC * NS
BPW = H // NW
CHUNK = 128
NCHUNK = BPW // CHUNK

_mesh = plsc.VectorSubcoreMesh(core_axis_name="c", subcore_axis_name="s")


@functools.partial(
    pl.kernel,
    mesh=_mesh,
    out_type=jax.ShapeDtypeStruct((H, D), jnp.float32),
    scratch_types=[
        pltpu.VMEM((BPW,), jnp.int32),
        pltpu.VMEM((NCHUNK, CHUNK, D), jnp.float32),
    ]
    + [pltpu.SemaphoreType.DMA] * (NCHUNK + 1),
)
def _sc_gather(t_hbm, table_hbm, out_hbm, idx_v, rows_v, *sems):
    gsems, ssem = sems[:NCHUNK], sems[NCHUNK]
    wid = lax.axis_index("s") * NC + lax.axis_index("c")
    base = wid * BPW
    pltpu.sync_copy(t_hbm.at[pl.ds(base, BPW)], idx_v)
    gcps = [
        pltpu.async_copy(
            table_hbm.at[idx_v.at[pl.ds(j * CHUNK, CHUNK)]], rows_v.at[j], gsems[j]
        )
        for j in range(NCHUNK)
    ]
    scps = []
    for j in range(NCHUNK):
        gcps[j].wait()
        scps.append(
            pltpu.async_copy(
                rows_v.at[j], out_hbm.at[pl.ds(base + j * CHUNK, CHUNK)], ssem
            )
        )
    for cp in scps:
        cp.wait()


V = 1000
VP = 1024
NSLAB = VP // D
BT = 4096
GT = (B - H) // BT
HB = H // BT


def _tc_body(t_ref, table_ref, out_ref):
    tb = t_ref[0, 0, :]
    a = tb >> 7
    c = tb & 127
    lanes = lax.broadcasted_iota(jnp.int32, (BT, D), 1)
    onehot_c = jnp.where(c[:, None] == lanes, 1.0, 0.0).astype(jnp.bfloat16)
    chunks = []
    for s in range(NSLAB):
        m = jnp.where(a[:, None] == s, 1.0, 0.0).astype(jnp.bfloat16)
        chunks.append(onehot_c * m)
    onehot = jnp.concatenate(chunks, axis=1)
    out_ref[...] = jnp.dot(
        onehot, table_ref[...], preferred_element_type=jnp.float32
    )


def _tc_lookup(t_hi, table):
    table_p = jnp.pad(table, ((0, VP - V), (0, 0))).astype(jnp.bfloat16)
    return pl.pallas_call(
        _tc_body,
        grid=(GT,),
        in_specs=[
            pl.BlockSpec((1, 1, BT), lambda i: (i, 0, 0)),
            pl.BlockSpec((VP, D), lambda i: (0, 0)),
        ],
        out_specs=pl.BlockSpec((BT, D), lambda i: (i + HB, 0)),
        out_shape=jax.ShapeDtypeStruct((B, D), jnp.float32),
    )(t_hi.reshape(GT, 1, BT), table_p)


def kernel(t, table):
    sc_rows = _sc_gather(t[:H], table)
    tc_out = _tc_lookup(t[H:], table)
    return lax.dynamic_update_slice(tc_out, sc_rows, (0, 0))


# speedup vs baseline: 1.0470x; 1.0470x over previous
import functools

import jax
import jax.numpy as jnp
from jax import lax
from jax.experimental import pallas as pl
from jax.experimental.pallas import tpu as pltpu
from jax.experimental.pallas import tpu_sc as plsc

B = 16384
D = 128
NC = 2
NS = 16
NW = NC * NS
BPW = B // NW
CHUNK = 256
NCHUNK = BPW // CHUNK

_mesh = plsc.VectorSubcoreMesh(core_axis_name="c", subcore_axis_name="s")


@functools.partial(
    pl.kernel,
    mesh=_mesh,
    out_type=jax.ShapeDtypeStruct((B, D), jnp.float32),
    scratch_types=[
        pltpu.VMEM((BPW,), jnp.int32),
        pltpu.VMEM((NCHUNK, CHUNK, D), jnp.float32),
    ]
    + [pltpu.SemaphoreType.DMA] * (NCHUNK + 1),
)
def _sc_gather(t_hbm, table_hbm, out_hbm, idx_v, rows_v, *sems):
    gsems, ssem = sems[:NCHUNK], sems[NCHUNK]
    wid = lax.axis_index("s") * NC + lax.axis_index("c")
    base = wid * BPW
    pltpu.sync_copy(t_hbm.at[pl.ds(base, BPW)], idx_v)
    gcps = [
        pltpu.async_copy(
            table_hbm.at[idx_v.at[pl.ds(j * CHUNK, CHUNK)]], rows_v.at[j], gsems[j]
        )
        for j in range(NCHUNK)
    ]
    scps = []
    for j in range(NCHUNK):
        gcps[j].wait()
        scps.append(
            pltpu.async_copy(
                rows_v.at[j], out_hbm.at[pl.ds(base + j * CHUNK, CHUNK)], ssem
            )
        )
    for cp in scps:
        cp.wait()


def kernel(t, table):
    return _sc_gather(t, table)


# speedup vs baseline: 1.1869x; 1.1336x over previous
import functools

import jax
import jax.numpy as jnp
from jax import lax
from jax.experimental import pallas as pl
from jax.experimental.pallas import tpu as pltpu
from jax.experimental.pallas import tpu_sc as plsc

B = 16384
D = 128
V = 1000
NC = 2
NS = 16
NW = NC * NS
BPW = B // NW
CHUNK = 256
NCHUNK = BPW // CHUNK

_mesh = plsc.VectorSubcoreMesh(core_axis_name="c", subcore_axis_name="s")


@functools.partial(
    pl.kernel,
    mesh=_mesh,
    out_type=jax.ShapeDtypeStruct((B, D), jnp.float32),
    scratch_types=[
        pltpu.VMEM((BPW,), jnp.int32),
        pltpu.VMEM((NCHUNK, CHUNK, D), jnp.float32),
        pltpu.VMEM_SHARED((V, D), jnp.float32),
    ]
    + [pltpu.SemaphoreType.DMA] * (NCHUNK + 1),
)
def _sc_gather2(t_hbm, table_hbm, out_hbm, idx_v, rows_v, tshared, *sems):
    gsems, ssem = sems[:NCHUNK], sems[NCHUNK]
    sid = lax.axis_index("s")
    wid = sid * NC + lax.axis_index("c")
    base = wid * BPW

    @pl.when(sid == 0)
    def _load_table():
        pltpu.sync_copy(table_hbm, tshared)

    pltpu.sync_copy(t_hbm.at[pl.ds(base, BPW)], idx_v)
    plsc.subcore_barrier()
    gcps = [
        pltpu.async_copy(
            tshared.at[idx_v.at[pl.ds(j * CHUNK, CHUNK)]], rows_v.at[j], gsems[j]
        )
        for j in range(NCHUNK)
    ]
    scps = []
    for j in range(NCHUNK):
        gcps[j].wait()
        scps.append(
            pltpu.async_copy(
                rows_v.at[j], out_hbm.at[pl.ds(base + j * CHUNK, CHUNK)], ssem
            )
        )
    for cp in scps:
        cp.wait()


def kernel(t, table):
    return _sc_gather2(t, table)
